# baseline (speedup 1.0000x reference)
import functools

import jax
import jax.numpy as jnp
from jax import lax
from jax.experimental import pallas as pl
from jax.experimental.pallas import tpu as pltpu
from jax.experimental.pallas import tpu_sc as plsc

_N = 10000
_NPAD = 10240
_E = 320000
_D = 128
_NC = 2
_NS = 16
_NW = _NC * _NS
_K = 96
_G = 16
_B = 2
_LA = _B - 1
_NG = 7
_TOTCH = _NW * _NG * _G
_EPAD = _TOTCH * _K
_ZR = 8


def _make_sc_agg(with_deg):
  mesh = plsc.VectorSubcoreMesh(core_axis_name="c", subcore_axis_name="s")
  rows_per_tile = _NPAD // _NS

  out_type = [jax.ShapeDtypeStruct((_NC, _NPAD, _D), jnp.float32)]
  scratch = [
      pltpu.VMEM((2, _G, _K), jnp.int32),
      pltpu.VMEM((2, _G, _K), jnp.int32),
      pltpu.VMEM((_B, _K, _D), jnp.float32),
      pltpu.VMEM((_ZR, _D), jnp.float32),
      pltpu.VMEM_SHARED((_NPAD, _D), jnp.float32),
      pltpu.SemaphoreType.DMA,
      pltpu.SemaphoreType.DMA,
      pltpu.SemaphoreType.DMA,
      pltpu.SemaphoreType.DMA,
      pltpu.SemaphoreType.DMA,
      pltpu.SemaphoreType.DMA,
      pltpu.SemaphoreType.DMA,
      pltpu.SemaphoreType.DMA,
  ]
  if with_deg:
    out_type.append(jax.ShapeDtypeStruct((_NW, 1, _NPAD), jnp.float32))
    scratch.append(pltpu.VMEM((1, _NPAD), jnp.float32))

  @functools.partial(pl.kernel, mesh=mesh, out_type=out_type,
                     scratch_types=scratch,
                     compiler_params=pltpu.CompilerParams(
                         needs_layout_passes=False))
  def k(x_hbm, src_hbm, dst_hbm, out_hbm, *rest):
    if with_deg:
      (deg_hbm, src_v, dst_v, rows_v, zb_v, acc_sh, isem,
       g0, g1, g2, s0, s1, s2, zsem, hist_v) = rest
    else:
      (src_v, dst_v, rows_v, zb_v, acc_sh, isem,
       g0, g1, g2, s0, s1, s2, zsem) = rest
    gsems = [g0, g1, g2]
    ssems = [s0, s1, s2]
    cid = lax.axis_index("c")
    sid = lax.axis_index("s")
    wid = cid * _NS + sid
    cbase = wid * (_NG * _G)

    coff = pl.multiple_of(cbase, _G)
    pltpu.async_copy(src_hbm.at[pl.ds(coff, _G)], src_v.at[0], isem)
    pltpu.async_copy(dst_hbm.at[pl.ds(coff, _G)], dst_v.at[0], isem)

    def zrow(i, carry):
      for j in range(_D // 16):
        zb_v[i, pl.ds(j * 16, 16)] = jnp.zeros((16,), jnp.float32)
      return carry

    lax.fori_loop(0, _ZR, zrow, 0)

    def zslice(i, carry):
      pltpu.async_copy(zb_v,
                       acc_sh.at[pl.ds(sid * rows_per_tile + i * _ZR, _ZR)],
                       zsem)
      return carry

    lax.fori_loop(0, rows_per_tile // _ZR, zslice, 0)

    def zdrain(i, carry):
      pltpu.make_async_copy(
          zb_v, acc_sh.at[pl.ds(sid * rows_per_tile + i * _ZR, _ZR)],
          zsem).wait()
      return carry

    lax.fori_loop(0, rows_per_tile // _ZR, zdrain, 0)

    if with_deg:
      def zhist(i, carry):
        hist_v[0, pl.ds(i * 16, 16)] = jnp.zeros((16,), jnp.float32)
        return carry

      lax.fori_loop(0, _NPAD // 16, zhist, 0)

    plsc.subcore_barrier()

    ones16 = jnp.full((16,), 1.0, jnp.float32)
    zero16 = jnp.zeros((16,), jnp.int32)

    def group(g, carry):
      gb = g % 2
      goff = pl.multiple_of(cbase + g * _G, _G)
      pltpu.make_async_copy(src_hbm.at[pl.ds(goff, _G)], src_v.at[gb],
                            isem).wait()
      pltpu.make_async_copy(dst_hbm.at[pl.ds(goff, _G)], dst_v.at[gb],
                            isem).wait()

      @pl.when(g + 1 < _NG)
      def _():
        goff2 = pl.multiple_of(cbase + (g + 1) * _G, _G)
        pltpu.async_copy(src_hbm.at[pl.ds(goff2, _G)], src_v.at[1 - gb],
                         isem)
        pltpu.async_copy(dst_hbm.at[pl.ds(goff2, _G)], dst_v.at[1 - gb],
                         isem)

      def gather(i, b):
        return pltpu.async_copy(x_hbm.at[src_v.at[gb, i]], rows_v.at[b],
                                gsems[b])

      def scatter(i, b):
        return pltpu.async_copy(rows_v.at[b], acc_sh.at[dst_v.at[gb, i]],
                                ssems[b], add=True)

      cps = [None] * _B
      scs = [None] * _B
      for j in range(_LA):
        cps[j] = gather(j, j)
      for i in range(_G):
        b = i % _B
        nb = (i + _LA) % _B
        if i + _LA < _G:
          if scs[nb] is not None:
            scs[nb].wait()
          cps[nb] = gather(i + _LA, nb)
        cps[b].wait()
        scs[b] = scatter(i, b)
        if with_deg:
          for j in range(_K // 16):
            idxv = dst_v[gb, i, pl.ds(j * 16, 16)]
            plsc.addupdate_scatter(hist_v, [zero16, idxv], ones16)
      for i in range(_G - _B, _G):
        scs[i % _B].wait()
      return carry

    lax.fori_loop(0, _NG, group, 0)

    plsc.subcore_barrier()

    pltpu.sync_copy(acc_sh.at[pl.ds(sid * rows_per_tile, rows_per_tile)],
                    out_hbm.at[cid, pl.ds(sid * rows_per_tile, rows_per_tile)])
    if with_deg:
      pltpu.sync_copy(hist_v, deg_hbm.at[wid])

  return k


_sc_agg_deg = _make_sc_agg(True)
_sc_agg = _make_sc_agg(False)

_R = 1000


def _dense1_body(p_ref, dT_ref, x_ref, wl_ref, b_ref, wr_ref, h_ref, rdeg_ref):
  s = p_ref[0] + p_ref[1]
  deg = jnp.sum(dT_ref[...], axis=1, keepdims=True)
  rdeg = 1.0 / jnp.maximum(deg, 1.0)
  m = jnp.dot(s, wl_ref[...], preferred_element_type=jnp.float32) * rdeg
  m = m + b_ref[...] + jnp.dot(x_ref[...], wr_ref[...],
                               preferred_element_type=jnp.float32)
  h_ref[...] = jnp.maximum(m, 0.0)
  rdeg_ref[...] = rdeg


def _dense2_body(p_ref, h_ref, wl_ref, b_ref, wr_ref, rdeg_ref, o_ref):
  s = p_ref[0] + p_ref[1]
  m = jnp.dot(s, wl_ref[...], preferred_element_type=jnp.float32)
  m = m * rdeg_ref[...]
  o_ref[...] = m + b_ref[...] + jnp.dot(h_ref[...], wr_ref[...],
                                        preferred_element_type=jnp.float32)


def _dense1(p, dT, x, wlT, b, wrT):
  grid = (_N // _R,)
  return pl.pallas_call(
      _dense1_body,
      grid=grid,
      in_specs=[
          pl.BlockSpec((_NC, _R, _D), lambda i: (0, i, 0)),
          pl.BlockSpec((_R, _NW), lambda i: (i, 0)),
          pl.BlockSpec((_R, _D), lambda i: (i, 0)),
          pl.BlockSpec((_D, _D), lambda i: (0, 0)),
          pl.BlockSpec((1, _D), lambda i: (0, 0)),
          pl.BlockSpec((_D, _D), lambda i: (0, 0)),
      ],
      out_specs=[
          pl.BlockSpec((_R, _D), lambda i: (i, 0)),
          pl.BlockSpec((_R, 1), lambda i: (i, 0)),
      ],
      out_shape=[
          jax.ShapeDtypeStruct((_N, _D), jnp.float32),
          jax.ShapeDtypeStruct((_N, 1), jnp.float32),
      ],
  )(p, dT, x, wlT, b, wrT)


def _dense2(p, h, wlT, b, wrT, rdeg):
  grid = (_N // _R,)
  return pl.pallas_call(
      _dense2_body,
      grid=grid,
      in_specs=[
          pl.BlockSpec((_NC, _R, _D), lambda i: (0, i, 0)),
          pl.BlockSpec((_R, _D), lambda i: (i, 0)),
          pl.BlockSpec((_D, _D), lambda i: (0, 0)),
          pl.BlockSpec((1, _D), lambda i: (0, 0)),
          pl.BlockSpec((_D, _D), lambda i: (0, 0)),
          pl.BlockSpec((_R, 1), lambda i: (i, 0)),
      ],
      out_specs=pl.BlockSpec((_R, _D), lambda i: (i, 0)),
      out_shape=jax.ShapeDtypeStruct((_N, _D), jnp.float32),
  )(p, h, wlT, b, wrT, rdeg)


@jax.jit
def kernel(x, edge_index, W_l1, b_l1, W_r1, W_l2, b_l2, W_r2):
  npd = _EPAD - _E
  pad_src = jnp.arange(npd, dtype=jnp.int32) % _N
  pad_dst = _N + jnp.arange(npd, dtype=jnp.int32) % (_NPAD - _N)
  src = jnp.concatenate([edge_index[0].astype(jnp.int32), pad_src]
                        ).reshape(_TOTCH, _K)
  dst = jnp.concatenate([edge_index[1].astype(jnp.int32), pad_dst]
                        ).reshape(_TOTCH, _K)
  p1, deg_p = _sc_agg_deg(x, src, dst)
  dT = deg_p[:, 0, :].T
  h, rdeg = _dense1(p1, dT, x, W_l1.T, b_l1.reshape(1, -1), W_r1.T)
  p2 = _sc_agg(h, src, dst)
  if isinstance(p2, (list, tuple)):
    p2 = p2[0]
  return _dense2(p2, h, W_l2.T, b_l2.reshape(1, -1), W_r2.T, rdeg)


# speedup vs baseline: 1.0431x; 1.0431x over previous
import functools

import jax
import jax.numpy as jnp
from jax import lax
from jax.experimental import pallas as pl
from jax.experimental.pallas import tpu as pltpu
from jax.experimental.pallas import tpu_sc as plsc

_N = 10000
_NPAD = 10240
_E = 320000
_D = 128
_NC = 2
_NS = 16
_NW = _NC * _NS
_K = 128
_G = 8
_B = 2
_LA = _B - 1
_NG = 10
_TOTCH = _NW * _NG * _G
_EPAD = _TOTCH * _K
_ZR = 8


def _make_sc_agg(with_deg):
  mesh = plsc.VectorSubcoreMesh(core_axis_name="c", subcore_axis_name="s")
  rows_per_tile = _NPAD // _NS

  out_type = [jax.ShapeDtypeStruct((_NC, _NPAD, _D), jnp.float32)]
  scratch = [
      pltpu.VMEM((2, _G, _K), jnp.int32),
      pltpu.VMEM((2, _G, _K), jnp.int32),
      pltpu.VMEM((_B, _K, _D), jnp.float32),
      pltpu.VMEM((_ZR, _D), jnp.float32),
      pltpu.VMEM_SHARED((_NPAD, _D), jnp.float32),
      pltpu.SemaphoreType.DMA,
      pltpu.SemaphoreType.DMA,
      pltpu.SemaphoreType.DMA,
      pltpu.SemaphoreType.DMA,
      pltpu.SemaphoreType.DMA,
      pltpu.SemaphoreType.DMA,
      pltpu.SemaphoreType.DMA,
      pltpu.SemaphoreType.DMA,
  ]
  if with_deg:
    out_type.append(jax.ShapeDtypeStruct((_NW, 1, _NPAD), jnp.float32))
    scratch.append(pltpu.VMEM((1, _NPAD), jnp.float32))

  @functools.partial(pl.kernel, mesh=mesh, out_type=out_type,
                     scratch_types=scratch,
                     compiler_params=pltpu.CompilerParams(
                         needs_layout_passes=False))
  def k(x_hbm, src_hbm, dst_hbm, out_hbm, *rest):
    if with_deg:
      (deg_hbm, src_v, dst_v, rows_v, zb_v, acc_sh, isem,
       g0, g1, g2, s0, s1, s2, zsem, hist_v) = rest
    else:
      (src_v, dst_v, rows_v, zb_v, acc_sh, isem,
       g0, g1, g2, s0, s1, s2, zsem) = rest
    gsems = [g0, g1, g2]
    ssems = [s0, s1, s2]
    cid = lax.axis_index("c")
    sid = lax.axis_index("s")
    wid = cid * _NS + sid
    cbase = wid * (_NG * _G)

    coff = pl.multiple_of(cbase, _G)
    pltpu.async_copy(src_hbm.at[pl.ds(coff, _G)], src_v.at[0], isem)
    pltpu.async_copy(dst_hbm.at[pl.ds(coff, _G)], dst_v.at[0], isem)

    def zrow(i, carry):
      for j in range(_D // 16):
        zb_v[i, pl.ds(j * 16, 16)] = jnp.zeros((16,), jnp.float32)
      return carry

    lax.fori_loop(0, _ZR, zrow, 0)

    def zslice(i, carry):
      pltpu.async_copy(zb_v,
                       acc_sh.at[pl.ds(sid * rows_per_tile + i * _ZR, _ZR)],
                       zsem)
      return carry

    lax.fori_loop(0, rows_per_tile // _ZR, zslice, 0)

    def zdrain(i, carry):
      pltpu.make_async_copy(
          zb_v, acc_sh.at[pl.ds(sid * rows_per_tile + i * _ZR, _ZR)],
          zsem).wait()
      return carry

    lax.fori_loop(0, rows_per_tile // _ZR, zdrain, 0)

    if with_deg:
      def zhist(i, carry):
        hist_v[0, pl.ds(i * 16, 16)] = jnp.zeros((16,), jnp.float32)
        return carry

      lax.fori_loop(0, _NPAD // 16, zhist, 0)

    plsc.subcore_barrier()

    ones16 = jnp.full((16,), 1.0, jnp.float32)
    zero16 = jnp.zeros((16,), jnp.int32)

    def group(g, carry):
      gb = g % 2
      goff = pl.multiple_of(cbase + g * _G, _G)
      pltpu.make_async_copy(src_hbm.at[pl.ds(goff, _G)], src_v.at[gb],
                            isem).wait()
      pltpu.make_async_copy(dst_hbm.at[pl.ds(goff, _G)], dst_v.at[gb],
                            isem).wait()

      @pl.when(g + 1 < _NG)
      def _():
        goff2 = pl.multiple_of(cbase + (g + 1) * _G, _G)
        pltpu.async_copy(src_hbm.at[pl.ds(goff2, _G)], src_v.at[1 - gb],
                         isem)
        pltpu.async_copy(dst_hbm.at[pl.ds(goff2, _G)], dst_v.at[1 - gb],
                         isem)

      def gather(i, b):
        return pltpu.async_copy(x_hbm.at[src_v.at[gb, i]], rows_v.at[b],
                                gsems[b])

      def scatter(i, b):
        return pltpu.async_copy(rows_v.at[b], acc_sh.at[dst_v.at[gb, i]],
                                ssems[b], add=True)

      cps = [None] * _B
      scs = [None] * _B
      for j in range(_LA):
        cps[j] = gather(j, j)
      for i in range(_G):
        b = i % _B
        nb = (i + _LA) % _B
        if i + _LA < _G:
          if scs[nb] is not None:
            scs[nb].wait()
          cps[nb] = gather(i + _LA, nb)
        cps[b].wait()
        scs[b] = scatter(i, b)
        if with_deg:
          for j in range(_K // 16):
            idxv = dst_v[gb, i, pl.ds(j * 16, 16)]
            plsc.addupdate_scatter(hist_v, [zero16, idxv], ones16)
      for i in range(_G - _B, _G):
        scs[i % _B].wait()
      return carry

    lax.fori_loop(0, _NG, group, 0)

    plsc.subcore_barrier()

    pltpu.sync_copy(acc_sh.at[pl.ds(sid * rows_per_tile, rows_per_tile)],
                    out_hbm.at[cid, pl.ds(sid * rows_per_tile, rows_per_tile)])
    if with_deg:
      pltpu.sync_copy(hist_v, deg_hbm.at[wid])

  return k


_sc_agg_deg = _make_sc_agg(True)
_sc_agg = _make_sc_agg(False)

_R = 1000


def _dense1_body(p_ref, dT_ref, x_ref, wl_ref, b_ref, wr_ref, h_ref, rdeg_ref):
  s = p_ref[0] + p_ref[1]
  deg = jnp.sum(dT_ref[...], axis=1, keepdims=True)
  rdeg = 1.0 / jnp.maximum(deg, 1.0)
  m = jnp.dot(s, wl_ref[...], preferred_element_type=jnp.float32) * rdeg
  m = m + b_ref[...] + jnp.dot(x_ref[...], wr_ref[...],
                               preferred_element_type=jnp.float32)
  h_ref[...] = jnp.maximum(m, 0.0)
  rdeg_ref[...] = rdeg


def _dense2_body(p_ref, h_ref, wl_ref, b_ref, wr_ref, rdeg_ref, o_ref):
  s = p_ref[0] + p_ref[1]
  m = jnp.dot(s, wl_ref[...], preferred_element_type=jnp.float32)
  m = m * rdeg_ref[...]
  o_ref[...] = m + b_ref[...] + jnp.dot(h_ref[...], wr_ref[...],
                                        preferred_element_type=jnp.float32)


def _dense1(p, dT, x, wlT, b, wrT):
  grid = (_N // _R,)
  return pl.pallas_call(
      _dense1_body,
      grid=grid,
      in_specs=[
          pl.BlockSpec((_NC, _R, _D), lambda i: (0, i, 0)),
          pl.BlockSpec((_R, _NW), lambda i: (i, 0)),
          pl.BlockSpec((_R, _D), lambda i: (i, 0)),
          pl.BlockSpec((_D, _D), lambda i: (0, 0)),
          pl.BlockSpec((1, _D), lambda i: (0, 0)),
          pl.BlockSpec((_D, _D), lambda i: (0, 0)),
      ],
      out_specs=[
          pl.BlockSpec((_R, _D), lambda i: (i, 0)),
          pl.BlockSpec((_R, 1), lambda i: (i, 0)),
      ],
      out_shape=[
          jax.ShapeDtypeStruct((_N, _D), jnp.float32),
          jax.ShapeDtypeStruct((_N, 1), jnp.float32),
      ],
  )(p, dT, x, wlT, b, wrT)


def _dense2(p, h, wlT, b, wrT, rdeg):
  grid = (_N // _R,)
  return pl.pallas_call(
      _dense2_body,
      grid=grid,
      in_specs=[
          pl.BlockSpec((_NC, _R, _D), lambda i: (0, i, 0)),
          pl.BlockSpec((_R, _D), lambda i: (i, 0)),
          pl.BlockSpec((_D, _D), lambda i: (0, 0)),
          pl.BlockSpec((1, _D), lambda i: (0, 0)),
          pl.BlockSpec((_D, _D), lambda i: (0, 0)),
          pl.BlockSpec((_R, 1), lambda i: (i, 0)),
      ],
      out_specs=pl.BlockSpec((_R, _D), lambda i: (i, 0)),
      out_shape=jax.ShapeDtypeStruct((_N, _D), jnp.float32),
  )(p, h, wlT, b, wrT, rdeg)


@jax.jit
def kernel(x, edge_index, W_l1, b_l1, W_r1, W_l2, b_l2, W_r2):
  npd = _EPAD - _E
  pad_src = jnp.arange(npd, dtype=jnp.int32) % _N
  pad_dst = _N + jnp.arange(npd, dtype=jnp.int32) % (_NPAD - _N)
  src = jnp.concatenate([edge_index[0].astype(jnp.int32), pad_src]
                        ).reshape(_TOTCH, _K)
  dst = jnp.concatenate([edge_index[1].astype(jnp.int32), pad_dst]
                        ).reshape(_TOTCH, _K)
  p1, deg_p = _sc_agg_deg(x, src, dst)
  dT = deg_p[:, 0, :].T
  h, rdeg = _dense1(p1, dT, x, W_l1.T, b_l1.reshape(1, -1), W_r1.T)
  p2 = _sc_agg(h, src, dst)
  if isinstance(p2, (list, tuple)):
    p2 = p2[0]
  return _dense2(p2, h, W_l2.T, b_l2.reshape(1, -1), W_r2.T, rdeg)
